# baseline (speedup 1.0000x reference)
import functools

import jax
import jax.numpy as jnp
from jax import lax
from jax.experimental import pallas as pl
from jax.experimental.pallas import tpu as pltpu
from jax.experimental.pallas import tpu_sc as plsc

V = 100000
D = 128
B = 1024
L = 50


_NC = 2
_NS = 16
_NW = _NC * _NS
_NROWS = B * L
_PER_W = _NROWS // _NW
_CHUNK = 80
_NCHUNK = _PER_W // _CHUNK
_BPW = B // _NW

_VB = (V + D - 1) // D


def _sc_gather(table, bias2d, idx_flat, lab, idxb):
    mesh = plsc.VectorSubcoreMesh(core_axis_name="c", subcore_axis_name="s")

    @functools.partial(
        pl.kernel,
        mesh=mesh,
        out_type=[
            jax.ShapeDtypeStruct((_NROWS, D), jnp.float32),
            jax.ShapeDtypeStruct((B, D), jnp.float32),
            jax.ShapeDtypeStruct((B, D), jnp.float32),
        ],
        scratch_types=[
            pltpu.VMEM((_CHUNK,), jnp.int32),
            pltpu.VMEM((_CHUNK, D), jnp.float32),
            pltpu.VMEM((_BPW,), jnp.int32),
            pltpu.VMEM((_BPW, D), jnp.float32),
            pltpu.SemaphoreType.DMA,
        ],
    )
    def k(table_hbm, bias_hbm, idx_hbm, lab_hbm, idxb_hbm,
          rows_hbm, lrows_hbm, brows_hbm,
          idx_v, rows_v, idxb_v, brows_v, sem):
        wid = lax.axis_index("s") * _NC + lax.axis_index("c")
        for c in range(_NCHUNK):
            base = wid * _PER_W + c * _CHUNK
            pltpu.sync_copy(idx_hbm.at[pl.ds(base, _CHUNK)], idx_v)
            pltpu.async_copy(table_hbm.at[idx_v], rows_v, sem).wait()
            pltpu.sync_copy(rows_v, rows_hbm.at[pl.ds(base, _CHUNK)])
        bbase = wid * _BPW
        pltpu.sync_copy(lab_hbm.at[pl.ds(bbase, _BPW)], idxb_v)
        pltpu.async_copy(table_hbm.at[idxb_v], brows_v, sem).wait()
        pltpu.sync_copy(brows_v, lrows_hbm.at[pl.ds(bbase, _BPW)])
        pltpu.sync_copy(idxb_hbm.at[pl.ds(bbase, _BPW)], idxb_v)
        pltpu.async_copy(bias_hbm.at[idxb_v], brows_v, sem).wait()
        pltpu.sync_copy(brows_v, brows_hbm.at[pl.ds(bbase, _BPW)])

    return k(table, bias2d, idx_flat, lab, idxb)


_BB = 128


def _attn_body(h_ref, a_ref, bt_ref, lab_ref, brow_ref, lane_ref,
               user_ref, ll_ref):
    hh = h_ref[...]
    hh2 = hh.reshape(L * _BB, D)
    t = jnp.tanh(
        lax.dot_general(hh2, a_ref[...], (((1,), (0,)), ((), ())),
                        preferred_element_type=jnp.float32))
    e_big = jnp.sum(t * bt_ref[...], axis=1, keepdims=True)
    e = jnp.concatenate(
        [e_big[l * _BB:(l + 1) * _BB, :] for l in range(L)], axis=1)
    m = jnp.max(e, axis=1, keepdims=True)
    p = jnp.exp(e - m)
    a = p / jnp.sum(p, axis=1, keepdims=True)
    acc = a[:, 0:1] * hh[0]
    for l in range(1, L):
        acc = acc + a[:, l:l + 1] * hh[l]
    user_ref[...] = acc
    lane = lax.broadcasted_iota(jnp.int32, (_BB, D), 1)
    lbias = jnp.sum(
        jnp.where(lane == lane_ref[...], brow_ref[...], 0.0),
        axis=1, keepdims=True)
    ll_ref[...] = jnp.sum(acc * lab_ref[...], axis=1, keepdims=True) + lbias


def _attention(h3, attn_a, attn_bt, labrows, brows, lanesel):
    grid = (B // _BB,)
    return pl.pallas_call(
        _attn_body,
        grid=grid,
        in_specs=[
            pl.BlockSpec((L, _BB, D), lambda i: (0, i, 0)),
            pl.BlockSpec((D, D), lambda i: (0, 0)),
            pl.BlockSpec((1, D), lambda i: (0, 0)),
            pl.BlockSpec((_BB, D), lambda i: (i, 0)),
            pl.BlockSpec((_BB, D), lambda i: (i, 0)),
            pl.BlockSpec((_BB, 1), lambda i: (i, 0)),
        ],
        out_specs=[
            pl.BlockSpec((_BB, D), lambda i: (i, 0)),
            pl.BlockSpec((_BB, 1), lambda i: (i, 0)),
        ],
        out_shape=[
            jax.ShapeDtypeStruct((B, D), jnp.float32),
            jax.ShapeDtypeStruct((B, 1), jnp.float32),
        ],
    )(h3, attn_a, attn_bt, labrows, brows, lanesel)


_BV = 1024
_NV = (V + _BV - 1) // _BV
_VPAD = _NV * _BV - V


def _logits_body(user_ref, tab_ref, bias_ref, ll_ref, ones_ref,
                 logits_ref, loss_ref, m_scr, s_scr):
    j = pl.program_id(0)

    @pl.when(j == 0)
    def _():
        m_scr[...] = jnp.full((1, B), -jnp.inf, jnp.float32)
        s_scr[...] = jnp.zeros((1, B), jnp.float32)

    u = user_ref[...]
    tab = tab_ref[...]
    logits = lax.dot_general(tab, u, (((1,), (1,)), ((), ())),
                             preferred_element_type=jnp.float32)
    logits = logits + lax.dot_general(
        bias_ref[...], ones_ref[...], (((0,), (0,)), ((), ())),
        preferred_element_type=jnp.float32)
    logits_ref[...] = logits

    def update(lm):
        tile_max = jnp.max(lm, axis=0, keepdims=True)
        m_old = m_scr[...]
        m_new = jnp.maximum(m_old, tile_max)
        p = jnp.exp(lm - m_new)
        psum = lax.dot_general(ones_ref[...], p, (((1,), (0,)), ((), ())),
                               preferred_element_type=jnp.float32)
        s_scr[...] = s_scr[...] * jnp.exp(m_old - m_new) + psum
        m_scr[...] = m_new

    @pl.when(j < _NV - 1)
    def _():
        update(logits)

    @pl.when(j == _NV - 1)
    def _():
        row = lax.broadcasted_iota(jnp.int32, (_BV, B), 0)
        update(jnp.where(row < _BV - _VPAD, logits, -1e30))

    @pl.when(j == _NV - 1)
    def _():
        lse = jnp.log(s_scr[...]) + m_scr[...]
        nll = lse - ll_ref[...]
        loss_ref[...] = jnp.sum(nll, axis=1, keepdims=True) / B


def _logits_loss(user, table, bias_pad, llT):
    assert _BV == B
    ones = jnp.ones((1, _BV), jnp.float32)
    return pl.pallas_call(
        _logits_body,
        grid=(_NV,),
        in_specs=[
            pl.BlockSpec((B, D), lambda j: (0, 0)),
            pl.BlockSpec((_BV, D), lambda j: (j, 0)),
            pl.BlockSpec((1, _BV), lambda j: (0, j)),
            pl.BlockSpec((1, B), lambda j: (0, 0)),
            pl.BlockSpec((1, _BV), lambda j: (0, 0)),
        ],
        out_specs=[
            pl.BlockSpec((_BV, B), lambda j: (j, 0)),
            pl.BlockSpec((1, 1), lambda j: (0, 0)),
        ],
        out_shape=[
            jax.ShapeDtypeStruct((V, B), jnp.float32),
            jax.ShapeDtypeStruct((1, 1), jnp.float32),
        ],
        scratch_shapes=[
            pltpu.VMEM((1, B), jnp.float32),
            pltpu.VMEM((1, B), jnp.float32),
        ],
    )(user, table, bias_pad, llT, ones)


def kernel(entity_ids, labels, token_embedding, attn_a, attn_b, rec_bias):
    ids = entity_ids.astype(jnp.int32)
    lab = labels.astype(jnp.int32)
    idx_flat = ids.T.reshape(_NROWS)
    bias2d = jnp.concatenate(
        [rec_bias, jnp.zeros((_VB * D - V,), jnp.float32)]).reshape(_VB, D)
    rows, labrows, brows = _sc_gather(
        token_embedding, bias2d, idx_flat, lab, lab // D)
    h3 = rows.reshape(L, B, D)
    user, ll = _attention(h3, attn_a, attn_b.reshape(1, D), labrows, brows,
                          (lab % D).reshape(B, 1))
    bias_pad = jnp.concatenate(
        [rec_bias, jnp.zeros((_VPAD,), jnp.float32)]).reshape(1, _NV * _BV)
    logitsT, loss = _logits_loss(user, token_embedding, bias_pad,
                                 ll.reshape(1, B))
    return (logitsT.T, labels, loss.reshape(()))


# speedup vs baseline: 1.3545x; 1.3545x over previous
import functools

import jax
import jax.numpy as jnp
from jax import lax
from jax.experimental import pallas as pl
from jax.experimental.pallas import tpu as pltpu
from jax.experimental.pallas import tpu_sc as plsc

V = 100000
D = 128
B = 1024
L = 50


_NC = 2
_NS = 16
_NW = _NC * _NS
_NROWS = B * L
_PER_W = _NROWS // _NW
_CHUNKS = [(c * 128, 128) for c in range(12)] + [(12 * 128, 64)]
_NCHUNK = len(_CHUNKS)
_BPW = B // _NW

_VB = (V + D - 1) // D


def _sc_gather(table, bias2d, idx2d, lab, idxb):
    mesh = plsc.VectorSubcoreMesh(core_axis_name="c", subcore_axis_name="s")

    @functools.partial(
        pl.kernel,
        mesh=mesh,
        out_type=[
            jax.ShapeDtypeStruct((_NROWS, D), jnp.float32),
            jax.ShapeDtypeStruct((B, D), jnp.float32),
            jax.ShapeDtypeStruct((B, D), jnp.float32),
        ],
        scratch_types=[
            pltpu.VMEM((_PER_W,), jnp.int32),
            pltpu.VMEM((128, D), jnp.float32),
            pltpu.VMEM((128, D), jnp.float32),
            pltpu.VMEM((_BPW,), jnp.int32),
            pltpu.VMEM((_BPW, D), jnp.float32),
            pltpu.SemaphoreType.DMA,
            pltpu.SemaphoreType.DMA,
            pltpu.SemaphoreType.DMA,
            pltpu.SemaphoreType.DMA,
        ],
    )
    def k(table_hbm, bias_hbm, idx_hbm, lab_hbm, idxb_hbm,
          rows_hbm, lrows_hbm, brows_hbm,
          idx_v, rows_v0, rows_v1, idxb_v, brows_v,
          g0, g1, o0, o1):
        wid = lax.axis_index("s") * _NC + lax.axis_index("c")
        rows_v = (rows_v0, rows_v1)
        gsem = (g0, g1)
        osem = (o0, o1)
        pltpu.sync_copy(idx_hbm.at[pl.ds(wid * _PER_W, _PER_W)], idx_v)
        gathers = [None] * _NCHUNK
        outs = [None] * _NCHUNK

        def start_gather(c):
            off, n = _CHUNKS[c]
            gathers[c] = pltpu.async_copy(
                table_hbm.at[idx_v.at[pl.ds(off, n)]],
                rows_v[c % 2].at[pl.ds(0, n)], gsem[c % 2])

        start_gather(0)
        for c in range(_NCHUNK):
            gathers[c].wait()
            off, n = _CHUNKS[c]
            base = wid * _PER_W + off
            outs[c] = pltpu.async_copy(
                rows_v[c % 2].at[pl.ds(0, n)],
                rows_hbm.at[pl.ds(base, n)], osem[c % 2])
            if c + 1 < _NCHUNK:
                if c >= 1:
                    outs[c - 1].wait()
                start_gather(c + 1)
        outs[_NCHUNK - 2].wait()
        outs[_NCHUNK - 1].wait()
        bbase = wid * _BPW
        pltpu.sync_copy(lab_hbm.at[pl.ds(bbase, _BPW)], idxb_v)
        pltpu.async_copy(table_hbm.at[idxb_v], brows_v, g0).wait()
        pltpu.sync_copy(brows_v, lrows_hbm.at[pl.ds(bbase, _BPW)])
        pltpu.sync_copy(idxb_hbm.at[pl.ds(bbase, _BPW)], idxb_v)
        pltpu.async_copy(bias_hbm.at[idxb_v], brows_v, g0).wait()
        pltpu.sync_copy(brows_v, brows_hbm.at[pl.ds(bbase, _BPW)])

    return k(table, bias2d, idx2d, lab, idxb)


_BB = 256


def _attn_body(h_ref, a_ref, bt_ref, lab_ref, brow_ref, lane_ref,
               user_ref, ll_ref):
    hh = h_ref[...]
    hh2 = hh.reshape(L * _BB, D)
    t = jnp.tanh(
        lax.dot_general(hh2, a_ref[...], (((1,), (0,)), ((), ())),
                        preferred_element_type=jnp.float32))
    e_big = jnp.sum(t * bt_ref[...], axis=1, keepdims=True)
    e = jnp.concatenate(
        [e_big[l * _BB:(l + 1) * _BB, :] for l in range(L)], axis=1)
    m = jnp.max(e, axis=1, keepdims=True)
    p = jnp.exp(e - m)
    a = p / jnp.sum(p, axis=1, keepdims=True)
    acc = a[:, 0:1] * hh[0]
    for l in range(1, L):
        acc = acc + a[:, l:l + 1] * hh[l]
    user_ref[...] = acc
    lane = lax.broadcasted_iota(jnp.int32, (_BB, D), 1)
    lbias = jnp.sum(
        jnp.where(lane == lane_ref[...], brow_ref[...], 0.0),
        axis=1, keepdims=True)
    ll_ref[...] = jnp.sum(acc * lab_ref[...], axis=1, keepdims=True) + lbias


def _attention(h3, attn_a, attn_bt, labrows, brows, lanesel):
    grid = (B // _BB,)
    return pl.pallas_call(
        _attn_body,
        grid=grid,
        in_specs=[
            pl.BlockSpec((L, _BB, D), lambda i: (0, i, 0)),
            pl.BlockSpec((D, D), lambda i: (0, 0)),
            pl.BlockSpec((1, D), lambda i: (0, 0)),
            pl.BlockSpec((_BB, D), lambda i: (i, 0)),
            pl.BlockSpec((_BB, D), lambda i: (i, 0)),
            pl.BlockSpec((_BB, 1), lambda i: (i, 0)),
        ],
        out_specs=[
            pl.BlockSpec((_BB, D), lambda i: (i, 0)),
            pl.BlockSpec((_BB, 1), lambda i: (i, 0)),
        ],
        out_shape=[
            jax.ShapeDtypeStruct((B, D), jnp.float32),
            jax.ShapeDtypeStruct((B, 1), jnp.float32),
        ],
    )(h3, attn_a, attn_bt, labrows, brows, lanesel)


_BV = 4096
_NV = (V + _BV - 1) // _BV
_VPAD = _NV * _BV - V
_NVPAD = 128


def _logits_body(user_ref, tab_ref, bias_ref, ll_ref, ones_ref,
                 logits_ref, loss_ref, m_scr, s_scr):
    j = pl.program_id(0)

    @pl.when(j == 0)
    def _():
        m_scr[...] = jnp.full((1, B), -jnp.inf, jnp.float32)
        s_scr[...] = jnp.zeros((1, B), jnp.float32)

    u = user_ref[...]
    tab = tab_ref[...]
    logits = lax.dot_general(tab, u, (((1,), (1,)), ((), ())),
                             preferred_element_type=jnp.float32)
    onehot = (lax.broadcasted_iota(jnp.int32, (_NVPAD, 1), 0) == j
              ).astype(jnp.float32)
    bias_col = lax.dot_general(bias_ref[...], onehot, (((1,), (0,)), ((), ())),
                               preferred_element_type=jnp.float32)
    logits = logits + bias_col
    logits_ref[...] = logits

    def update(lm, first):
        tile_max = jnp.max(lm, axis=0, keepdims=True)
        m_old = m_scr[...]
        m_new = jnp.maximum(m_old, tile_max)
        if first:
            p = jnp.exp(lm - m_new)
            s_scr[...] = lax.dot_general(
                ones_ref[...], p, (((1,), (0,)), ((), ())),
                preferred_element_type=jnp.float32)
        else:
            p = jnp.exp(lm - m_old)
            psum = lax.dot_general(
                ones_ref[...], p, (((1,), (0,)), ((), ())),
                preferred_element_type=jnp.float32)
            s_scr[...] = (s_scr[...] + psum) * jnp.exp(m_old - m_new)
        m_scr[...] = m_new

    @pl.when(j == 0)
    def _():
        update(logits, True)

    @pl.when(jnp.logical_and(j > 0, j < _NV - 1))
    def _():
        update(logits, False)

    @pl.when(j == _NV - 1)
    def _():
        row = lax.broadcasted_iota(jnp.int32, (_BV, B), 0)
        update(jnp.where(row < _BV - _VPAD, logits, -1e30), False)

    @pl.when(j == _NV - 1)
    def _():
        lse = jnp.log(s_scr[...]) + m_scr[...]
        nll = lse - ll_ref[...]
        loss_ref[...] = jnp.sum(nll, axis=1, keepdims=True) / B


def _logits_loss(user, table, bias_pad, llT):
    ones = jnp.ones((1, _BV), jnp.float32)
    return pl.pallas_call(
        _logits_body,
        grid=(_NV,),
        in_specs=[
            pl.BlockSpec((B, D), lambda j: (0, 0)),
            pl.BlockSpec((_BV, D), lambda j: (j, 0)),
            pl.BlockSpec((_BV, _NVPAD), lambda j: (0, 0)),
            pl.BlockSpec((1, B), lambda j: (0, 0)),
            pl.BlockSpec((1, _BV), lambda j: (0, 0)),
        ],
        out_specs=[
            pl.BlockSpec((_BV, B), lambda j: (j, 0)),
            pl.BlockSpec((1, 1), lambda j: (0, 0)),
        ],
        out_shape=[
            jax.ShapeDtypeStruct((V, B), jnp.float32),
            jax.ShapeDtypeStruct((1, 1), jnp.float32),
        ],
        scratch_shapes=[
            pltpu.VMEM((1, B), jnp.float32),
            pltpu.VMEM((1, B), jnp.float32),
        ],
    )(user, table, bias_pad, llT, ones)


def kernel(entity_ids, labels, token_embedding, attn_a, attn_b, rec_bias):
    ids = entity_ids.astype(jnp.int32)
    lab = labels.astype(jnp.int32)
    idx2d = ids.T.reshape(_NROWS)
    bias2d = jnp.concatenate(
        [rec_bias, jnp.zeros((_VB * D - V,), jnp.float32)]).reshape(_VB, D)
    rows, labrows, brows = _sc_gather(
        token_embedding, bias2d, idx2d, lab, lab // D)
    h3 = rows.reshape(L, B, D)
    user, ll = _attention(h3, attn_a, attn_b.reshape(1, D), labrows, brows,
                          (lab % D).reshape(B, 1))
    bias_cols = jnp.zeros((_BV, _NVPAD), jnp.float32).at[:, :_NV].set(
        jnp.concatenate(
            [rec_bias, jnp.zeros((_VPAD,), jnp.float32)]).reshape(_NV, _BV).T)
    logitsT, loss = _logits_loss(user, token_embedding, bias_cols,
                                 ll.reshape(1, B))
    return (logitsT.T, labels, loss.reshape(()))
